# Initial kernel scaffold; baseline (speedup 1.0000x reference)
import jax
import jax.numpy as jnp
from jax.experimental import pallas as pl


def kernel(x, edge_index, edge_attr, W, b):
    raise NotImplementedError("write your pallas kernel here")


# speedup vs baseline: 4.2445x; 4.2445x over previous
import functools

import jax
import jax.numpy as jnp
from jax import lax
from jax.experimental import pallas as pl
from jax.experimental.pallas import tpu as pltpu
from jax.experimental.pallas import tpu_sc as plsc

_NC = 2
_NS = 16
_CHUNK = 128


def _make_sc_agg(N, D, E):
    NW = _NC * _NS
    ept = ((-(-E // NW) + _CHUNK - 1) // _CHUNK) * _CHUNK
    n_chunks = ept // _CHUNK
    e_pad = ept * NW
    n_acc = -(-(N + 1) // (_NS * 8)) * (_NS * 8)
    rpt = n_acc // _NS

    mesh = plsc.VectorSubcoreMesh(core_axis_name="c", subcore_axis_name="s")

    @functools.partial(
        pl.kernel,
        out_type=jax.ShapeDtypeStruct((_NC, n_acc, D), jnp.float32),
        mesh=mesh,
        scratch_types=[
            pltpu.VMEM((_CHUNK,), jnp.int32),
            pltpu.VMEM((_CHUNK,), jnp.int32),
            pltpu.VMEM((_CHUNK, D), jnp.float32),
            pltpu.VMEM_SHARED((n_acc, D), jnp.float32),
            pltpu.SemaphoreType.DMA,
        ],
    )
    def agg(x_hbm, row_hbm, col_hbm, zero_hbm, out_hbm, col_v, row_v,
            rows_v, acc, sem):
        c = lax.axis_index("c")
        s = lax.axis_index("s")
        w = c * _NS + s
        pltpu.sync_copy(zero_hbm, acc.at[pl.ds(s * rpt, rpt)])
        plsc.subcore_barrier()
        base0 = w * ept

        @pl.loop(0, n_chunks)
        def _edge_chunk(k):
            base = base0 + k * _CHUNK
            pltpu.sync_copy(col_hbm.at[pl.ds(base, _CHUNK)], col_v)
            pltpu.sync_copy(row_hbm.at[pl.ds(base, _CHUNK)], row_v)
            pltpu.async_copy(x_hbm.at[col_v], rows_v, sem).wait()
            pltpu.sync_copy(rows_v, acc.at[row_v], add=True)

        plsc.subcore_barrier()
        pltpu.sync_copy(acc.at[pl.ds(s * rpt, rpt)],
                        out_hbm.at[c, pl.ds(s * rpt, rpt)])

    return agg, e_pad, n_acc


def _tc_linear_body(x_ref, p0_ref, p1_ref, w_ref, b_ref, o_ref):
    s = x_ref[...] + p0_ref[...] + p1_ref[...]
    h = lax.dot_general(s, w_ref[...], (((1,), (1,)), ((), ())),
                        preferred_element_type=jnp.float32)
    h = h + b_ref[...]
    o_ref[...] = h * jax.nn.sigmoid(h)


def kernel(x, edge_index, edge_attr, W, b):
    N, D = x.shape
    E = edge_index.shape[1]
    ei = edge_index.astype(jnp.int32)
    row, col = ei[0], ei[1]

    agg_fn, e_pad, n_acc = _make_sc_agg(N, D, E)
    pad = e_pad - E
    row_p = jnp.concatenate([row, jnp.full((pad,), N, jnp.int32)])
    col_p = jnp.concatenate([col, jnp.zeros((pad,), jnp.int32)])
    zeros = jnp.zeros((n_acc // _NS, D), jnp.float32)

    parts = agg_fn(x, row_p, col_p, zeros)
    p0 = parts[0, :N]
    p1 = parts[1, :N]

    RB = 1000
    return pl.pallas_call(
        _tc_linear_body,
        grid=(N // RB,),
        in_specs=[
            pl.BlockSpec((RB, D), lambda i: (i, 0)),
            pl.BlockSpec((RB, D), lambda i: (i, 0)),
            pl.BlockSpec((RB, D), lambda i: (i, 0)),
            pl.BlockSpec((D, D), lambda i: (0, 0)),
            pl.BlockSpec((1, D), lambda i: (0, 0)),
        ],
        out_specs=pl.BlockSpec((RB, D), lambda i: (i, 0)),
        out_shape=jax.ShapeDtypeStruct((N, D), jnp.float32),
    )(x, p0, p1, W, b.reshape(1, D))
